# baseline (speedup 1.0000x reference)
import functools

import jax
import jax.numpy as jnp
from jax import lax
from jax.experimental import pallas as pl
from jax.experimental.pallas import tpu as pltpu
from jax.experimental.pallas import tpu_sc as plsc

_VOCAB = 1000000
_HIDDEN = 64
_SEQ = 200
_BATCH = 4096

_NC = 2
_NS = 16
_NW = _NC * _NS
_SEQ_PER_W = _BATCH // _NW

_G0 = 128
_G1 = _SEQ - _G0


def _body(tok_hbm, emb_hbm, pos_hbm, out_hbm, pos_v, idx_v, rows_v, sem):
    c = lax.axis_index("c")
    s = lax.axis_index("s")
    wid = s * _NC + c

    pltpu.sync_copy(pos_hbm, pos_v)

    def seq_body(i, carry):
        base = (wid * _SEQ_PER_W + i) * _SEQ

        pltpu.sync_copy(tok_hbm.at[pl.ds(base, _SEQ)], idx_v)

        cp0 = pltpu.async_copy(
            emb_hbm.at[idx_v.at[pl.ds(0, _G0)]], rows_v.at[pl.ds(0, _G0)], sem
        )
        cp1 = pltpu.async_copy(
            emb_hbm.at[idx_v.at[pl.ds(_G0, _G1)]], rows_v.at[pl.ds(_G0, _G1)], sem
        )
        cp0.wait()
        cp1.wait()

        def l_body(l, c2):
            for g in range(_HIDDEN // 16):
                sl = pl.ds(g * 16, 16)
                rows_v[l, sl] = rows_v[l, sl] + pos_v[l, sl]
            return c2

        lax.fori_loop(0, _SEQ, l_body, 0)

        pltpu.sync_copy(rows_v, out_hbm.at[pl.ds(base, _SEQ)])
        return carry

    lax.fori_loop(0, _SEQ_PER_W, seq_body, 0)


@functools.partial(jax.jit, static_argnames=())
def _emb_lookup(tokens_flat, emb_table, pos_table):
    kfn = pl.kernel(
        _body,
        mesh=plsc.VectorSubcoreMesh(core_axis_name="c", subcore_axis_name="s"),
        out_type=jax.ShapeDtypeStruct((_BATCH * _SEQ, _HIDDEN), jnp.float32),
        scratch_types=[
            pltpu.VMEM((_SEQ, _HIDDEN), jnp.float32),
            pltpu.VMEM((_SEQ,), jnp.int32),
            pltpu.VMEM((_SEQ, _HIDDEN), jnp.float32),
            pltpu.SemaphoreType.DMA,
        ],
        compiler_params=pltpu.CompilerParams(use_tc_tiling_on_sc=False),
    )
    return kfn(tokens_flat, emb_table, pos_table)


def kernel(tokens, emb_table, pos_table):
    batch, seq = tokens.shape
    out = _emb_lookup(tokens.reshape(-1), emb_table, pos_table)
    return out.reshape(batch, seq, emb_table.shape[1])


# speedup vs baseline: 1.1539x; 1.1539x over previous
import functools

import jax
import jax.numpy as jnp
from jax import lax
from jax.experimental import pallas as pl
from jax.experimental.pallas import tpu as pltpu
from jax.experimental.pallas import tpu_sc as plsc

_VOCAB = 1000000
_HIDDEN = 64
_SEQ = 200
_BATCH = 4096

_NC = 2
_NS = 16
_NW = _NC * _NS
_SEQ_PER_W = _BATCH // _NW

_G0 = 128
_G1 = _SEQ - _G0
_ROW_BYTES = _HIDDEN * 4
_UNROLL = 4


def _body(tok_hbm, emb_hbm, pos_hbm, out_hbm, pos_v,
          idx0, idx1, rows0, rows1, si0, si1, sg0, sg1, so0, so1):
    c = lax.axis_index("c")
    s = lax.axis_index("s")
    wid = s * _NC + c

    idx = (idx0, idx1)
    rows = (rows0, rows1)
    si = (si0, si1)
    sg = (sg0, sg1)
    so = (so0, so1)

    def base_of(i):
        return (wid * _SEQ_PER_W + i) * _SEQ

    def fire_idx(i, b):
        pltpu.async_copy(tok_hbm.at[pl.ds(base_of(i), _SEQ)], idx[b], si[b])

    def wait_idx(b):
        pltpu.make_async_copy(
            tok_hbm.at[pl.ds(0, _SEQ)], idx[b], si[b]).wait()

    def fire_gather(b):
        pltpu.async_copy(
            emb_hbm.at[idx[b].at[pl.ds(0, _G0)]], rows[b].at[pl.ds(0, _G0)],
            sg[b])
        pltpu.async_copy(
            emb_hbm.at[idx[b].at[pl.ds(_G0, _G1)]],
            rows[b].at[pl.ds(_G0, _G1)], sg[b])

    def wait_gather(b):
        pltpu.make_async_copy(
            out_hbm.at[pl.ds(0, _SEQ)], rows[b], sg[b]).wait()

    def fire_wb(i, b):
        pltpu.async_copy(rows[b], out_hbm.at[pl.ds(base_of(i), _SEQ)], so[b])

    def wait_wb(b):
        pltpu.make_async_copy(
            out_hbm.at[pl.ds(0, _SEQ)], rows[b], so[b]).wait()

    pltpu.sync_copy(pos_hbm, pos_v)

    fire_idx(0, 0)
    fire_idx(1, 1)
    wait_idx(0)
    fire_gather(0)

    def seq_body(i, carry):
        b = lax.rem(i, 2)

        def for_buf(bb):
            nb = 1 - bb

            @pl.when(i >= 1)
            def _():
                wait_wb(nb)

            @pl.when(i + 1 < _SEQ_PER_W)
            def _():
                wait_idx(nb)
                fire_gather(nb)

            wait_gather(bb)

            def l_body(j, c2):
                for u in range(_UNROLL):
                    l = j * _UNROLL + u
                    for g in range(_HIDDEN // 16):
                        sl = pl.ds(g * 16, 16)
                        rows[bb][l, sl] = rows[bb][l, sl] + pos_v[l, sl]
                return c2

            lax.fori_loop(0, _SEQ // _UNROLL, l_body, 0)

            fire_wb(i, bb)

            @pl.when(i + 2 < _SEQ_PER_W)
            def _():
                fire_idx(i + 2, bb)

        @pl.when(b == 0)
        def _():
            for_buf(0)

        @pl.when(b == 1)
        def _():
            for_buf(1)

        return carry

    lax.fori_loop(0, _SEQ_PER_W, seq_body, 0)

    wait_wb((_SEQ_PER_W - 1) % 2)


@functools.partial(jax.jit, static_argnames=())
def _emb_lookup(tokens_flat, emb_table, pos_table):
    kfn = pl.kernel(
        _body,
        mesh=plsc.VectorSubcoreMesh(core_axis_name="c", subcore_axis_name="s"),
        out_type=jax.ShapeDtypeStruct((_BATCH * _SEQ, _HIDDEN), jnp.float32),
        scratch_types=[
            pltpu.VMEM((_SEQ, _HIDDEN), jnp.float32),
            pltpu.VMEM((_SEQ,), jnp.int32),
            pltpu.VMEM((_SEQ,), jnp.int32),
            pltpu.VMEM((_SEQ, _HIDDEN), jnp.float32),
            pltpu.VMEM((_SEQ, _HIDDEN), jnp.float32),
            pltpu.SemaphoreType.DMA,
            pltpu.SemaphoreType.DMA,
            pltpu.SemaphoreType.DMA,
            pltpu.SemaphoreType.DMA,
            pltpu.SemaphoreType.DMA,
            pltpu.SemaphoreType.DMA,
        ],
        compiler_params=pltpu.CompilerParams(use_tc_tiling_on_sc=False),
    )
    return kfn(tokens_flat, emb_table, pos_table)


def kernel(tokens, emb_table, pos_table):
    batch, seq = tokens.shape
    out = _emb_lookup(tokens.reshape(-1), emb_table, pos_table)
    return out.reshape(batch, seq, emb_table.shape[1])
